# baseline (speedup 1.0000x reference)
import functools

import jax
import jax.numpy as jnp
from jax import lax
from jax.experimental import pallas as pl
from jax.experimental.pallas import tpu as pltpu
from jax.experimental.pallas import tpu_sc as plsc

N = 10000
D = 128
E = 320000

NC = 2
NS = 16
NW = NC * NS
CH = 128
GRP = 8
NG0 = 20
NG1 = 0
E0 = NS * NG0 * GRP * CH
E1 = NS * NG1 * GRP * CH
EPAD = E0 + E1 - E
ROWA = 624
ROWL = N - 15 * ROWA


@functools.lru_cache(maxsize=None)
def _mesh():
    return plsc.VectorSubcoreMesh(core_axis_name="c", subcore_axis_name="s",
                                  num_cores=NC, num_subcores=NS)


def _sc_agg_body(h_hbm, srcA_hbm, dstA_hbm, srcB_hbm, dstB_hbm, p_hbm,
                 src_g, dst_g, rowsA, rowsB, semA, semB, acc_h):
    c = lax.axis_index("c")
    s = lax.axis_index("s")

    zv = jnp.zeros((16,), jnp.float32)

    @pl.loop(0, CH)
    def _(i):
        for j in range(D // 16):
            rowsA[i, pl.ds(j * 16, 16)] = zv

    @pl.loop(0, 4)
    def _(k):
        pltpu.sync_copy(rowsA, acc_h.at[pl.ds(s * ROWA + k * CH, CH)])
    pltpu.sync_copy(rowsA.at[pl.ds(0, ROWA - 4 * CH)],
                    acc_h.at[pl.ds(s * ROWA + 4 * CH, ROWA - 4 * CH)])

    @pl.when(s == NS - 1)
    def _():
        pltpu.sync_copy(rowsA.at[pl.ds(0, ROWL + 8 - ROWA)],
                        acc_h.at[pl.ds(16 * ROWA, ROWL + 8 - ROWA)])

    plsc.subcore_barrier()

    def start(idx_row, buf, sem):
        pltpu.async_copy(h_hbm.at[idx_row], buf, sem)

    def wait(buf, sem):
        pltpu.make_async_copy(h_hbm.at[pl.ds(0, CH)], buf, sem).wait()

    bufs = (rowsA, rowsB)
    sems = (semA, semB)

    def run_edges(src_hbm, dst_hbm, ngroups):
        @pl.loop(0, ngroups)
        def _(g):
            pltpu.sync_copy(src_hbm.at[s, g], src_g)
            pltpu.sync_copy(dst_hbm.at[s, g], dst_g)
            start(src_g.at[0], bufs[0], sems[0])
            for r in range(1, GRP):
                start(src_g.at[r], bufs[r % 2], sems[r % 2])
                wait(bufs[(r - 1) % 2], sems[(r - 1) % 2])
                pltpu.sync_copy(bufs[(r - 1) % 2], acc_h.at[dst_g.at[r - 1]],
                                add=True)
            wait(bufs[(GRP - 1) % 2], sems[(GRP - 1) % 2])
            pltpu.sync_copy(bufs[(GRP - 1) % 2], acc_h.at[dst_g.at[GRP - 1]],
                            add=True)

    @pl.when(c == 0)
    def _():
        run_edges(srcA_hbm, dstA_hbm, NG0)

    @pl.when(c == 1)
    def _():
        run_edges(srcB_hbm, dstB_hbm, NG1)

    plsc.subcore_barrier()

    @pl.when(s < NS - 1)
    def _():
        pltpu.sync_copy(acc_h.at[pl.ds(s * ROWA, ROWA)],
                        p_hbm.at[c, pl.ds(s * ROWA, ROWA)])

    @pl.when(s == NS - 1)
    def _():
        pltpu.sync_copy(acc_h.at[pl.ds(15 * ROWA, ROWL)],
                        p_hbm.at[c, pl.ds(15 * ROWA, ROWL)])


@functools.lru_cache(maxsize=None)
def _sc_agg():
    return pl.kernel(
        _sc_agg_body,
        out_type=jax.ShapeDtypeStruct((NC, N, D), jnp.float32),
        mesh=_mesh(),
        scratch_types=(
            pltpu.VMEM((GRP, CH), jnp.int32),
            pltpu.VMEM((GRP, CH), jnp.int32),
            pltpu.VMEM((CH, D), jnp.float32),
            pltpu.VMEM((CH, D), jnp.float32),
            pltpu.SemaphoreType.DMA,
            pltpu.SemaphoreType.DMA,
            pltpu.VMEM_SHARED((N + 8, D), jnp.float32),
        ),
    )


def _tc_dense_body(bn, p_ref, c_ref, h_ref, wl_ref, bl_ref, wr_ref,
                   g_ref, b_ref, out_ref):
    cnt = c_ref[0, :, 0:1] + c_ref[1, :, 0:1]
    inv = 1.0 / jnp.maximum(cnt, 1.0)
    agg = (p_ref[0] + p_ref[1]) * inv
    z = (jnp.dot(agg, wl_ref[...], preferred_element_type=jnp.float32)
         + bl_ref[...]
         + jnp.dot(h_ref[...], wr_ref[...], preferred_element_type=jnp.float32))
    if bn:
        mu = jnp.mean(z, axis=0, keepdims=True)
        var = jnp.mean((z - mu) ** 2, axis=0, keepdims=True)
        z = (z - mu) * lax.rsqrt(var + 1e-5) * g_ref[...] + b_ref[...]
        z = jnp.maximum(z, 0.0)
    out_ref[...] = z


def _tc_dense(bn, p, ccnt, h, wl, bl, wr, g, b):
    return pl.pallas_call(
        functools.partial(_tc_dense_body, bn),
        out_shape=jax.ShapeDtypeStruct((N, D), jnp.float32),
    )(p, ccnt, h, wl, bl.reshape(1, D), wr, g.reshape(1, D), b.reshape(1, D))


def kernel(x, edge_index, W_l0, b_l0, W_r0, W_l1, b_l1, W_r1, W_l2, b_l2,
           W_r2, bn_g0, bn_b0, bn_g1, bn_b1):
    src = edge_index[0].astype(jnp.int32)
    dst = edge_index[1].astype(jnp.int32)
    srcp = jnp.concatenate([src, jnp.zeros((EPAD,), jnp.int32)])
    dstp = jnp.concatenate([dst, jnp.full((EPAD,), N, jnp.int32)])
    srcA = srcp[:E0].reshape(NS, NG0, GRP, CH)
    dstA = dstp[:E0].reshape(NS, NG0, GRP, CH)
    if NG1 > 0:
        srcB = srcp[E0:].reshape(NS, NG1, GRP, CH)
        dstB = dstp[E0:].reshape(NS, NG1, GRP, CH)
    else:
        srcB = jnp.zeros((NS, 1, GRP, CH), jnp.int32)
        dstB = jnp.full((NS, 1, GRP, CH), N, jnp.int32)
    ones_nd = jnp.ones((N, D), jnp.float32)

    agg = _sc_agg()
    ccnt = agg(ones_nd, srcA, dstA, srcB, dstB)
    p0 = agg(x, srcA, dstA, srcB, dstB)
    h1 = _tc_dense(True, p0, ccnt, x, W_l0, b_l0, W_r0, bn_g0, bn_b0)
    p1 = agg(h1, srcA, dstA, srcB, dstB)
    h2 = _tc_dense(True, p1, ccnt, h1, W_l1, b_l1, W_r1, bn_g1, bn_b1)
    p2 = agg(h2, srcA, dstA, srcB, dstB)
    h3 = _tc_dense(False, p2, ccnt, h2, W_l2, b_l2, W_r2, bn_g0, bn_b0)
    return h3


# speedup vs baseline: 3.8758x; 3.8758x over previous
import functools

import jax
import jax.numpy as jnp
from jax import lax
from jax.experimental import pallas as pl
from jax.experimental.pallas import tpu as pltpu
from jax.experimental.pallas import tpu_sc as plsc

N = 10000
D = 128
E = 320000

NC = 2
NS = 16
NW = NC * NS
CH = 128
GRP = 8
NG0 = 10
NG1 = 10
E0 = NS * NG0 * GRP * CH
E1 = NS * NG1 * GRP * CH
EPAD = E0 + E1 - E
PADR = 512
ROWA = 624
ROWL = N - 15 * ROWA


@functools.lru_cache(maxsize=None)
def _mesh():
    return plsc.VectorSubcoreMesh(core_axis_name="c", subcore_axis_name="s",
                                  num_cores=NC, num_subcores=NS)


def _sc_agg_body(h_hbm, srcA_hbm, dstA_hbm, srcB_hbm, dstB_hbm, p_hbm,
                 src_g, dst_g, rowsA, rowsB, semA, semB, acc_h):
    c = lax.axis_index("c")
    s = lax.axis_index("s")

    zv = jnp.zeros((16,), jnp.float32)

    @pl.loop(0, CH)
    def _(i):
        for j in range(D // 16):
            rowsA[i, pl.ds(j * 16, 16)] = zv

    @pl.loop(0, 4)
    def _(k):
        pltpu.sync_copy(rowsA, acc_h.at[pl.ds(s * ROWA + k * CH, CH)])
    pltpu.sync_copy(rowsA.at[pl.ds(0, ROWA - 4 * CH)],
                    acc_h.at[pl.ds(s * ROWA + 4 * CH, ROWA - 4 * CH)])
    pltpu.sync_copy(rowsA.at[pl.ds(0, PADR // NS)],
                    acc_h.at[pl.ds(N + s * (PADR // NS), PADR // NS)])

    @pl.when(s == NS - 1)
    def _():
        pltpu.sync_copy(rowsA.at[pl.ds(0, N - 16 * ROWA)],
                        acc_h.at[pl.ds(16 * ROWA, N - 16 * ROWA)])

    plsc.subcore_barrier()

    def start(idx_row, buf, sem):
        pltpu.async_copy(h_hbm.at[idx_row], buf, sem)

    def wait(buf, sem):
        pltpu.make_async_copy(h_hbm.at[pl.ds(0, CH)], buf, sem).wait()

    bufs = (rowsA, rowsB)
    sems = (semA, semB)

    def run_edges(src_hbm, dst_hbm, ngroups):
        @pl.loop(0, ngroups)
        def _(g):
            pltpu.sync_copy(src_hbm.at[s, g], src_g)
            pltpu.sync_copy(dst_hbm.at[s, g], dst_g)
            start(src_g.at[0], bufs[0], sems[0])
            for r in range(1, GRP):
                start(src_g.at[r], bufs[r % 2], sems[r % 2])
                wait(bufs[(r - 1) % 2], sems[(r - 1) % 2])
                pltpu.sync_copy(bufs[(r - 1) % 2], acc_h.at[dst_g.at[r - 1]],
                                add=True)
            wait(bufs[(GRP - 1) % 2], sems[(GRP - 1) % 2])
            pltpu.sync_copy(bufs[(GRP - 1) % 2], acc_h.at[dst_g.at[GRP - 1]],
                            add=True)

    @pl.when(c == 0)
    def _():
        run_edges(srcA_hbm, dstA_hbm, NG0)

    @pl.when(c == 1)
    def _():
        run_edges(srcB_hbm, dstB_hbm, NG1)

    plsc.subcore_barrier()

    @pl.when(s < NS - 1)
    def _():
        pltpu.sync_copy(acc_h.at[pl.ds(s * ROWA, ROWA)],
                        p_hbm.at[c, pl.ds(s * ROWA, ROWA)])

    @pl.when(s == NS - 1)
    def _():
        pltpu.sync_copy(acc_h.at[pl.ds(15 * ROWA, ROWL)],
                        p_hbm.at[c, pl.ds(15 * ROWA, ROWL)])


@functools.lru_cache(maxsize=None)
def _sc_agg():
    return pl.kernel(
        _sc_agg_body,
        out_type=jax.ShapeDtypeStruct((NC, N, D), jnp.float32),
        mesh=_mesh(),
        scratch_types=(
            pltpu.VMEM((GRP, CH), jnp.int32),
            pltpu.VMEM((GRP, CH), jnp.int32),
            pltpu.VMEM((CH, D), jnp.float32),
            pltpu.VMEM((CH, D), jnp.float32),
            pltpu.SemaphoreType.DMA,
            pltpu.SemaphoreType.DMA,
            pltpu.VMEM_SHARED((N + PADR, D), jnp.float32),
        ),
    )


def _tc_dense_body(bn, p_ref, c_ref, h_ref, wl_ref, bl_ref, wr_ref,
                   g_ref, b_ref, out_ref):
    cnt = c_ref[0, :, 0:1] + c_ref[1, :, 0:1]
    inv = 1.0 / jnp.maximum(cnt, 1.0)
    agg = (p_ref[0] + p_ref[1]) * inv
    z = (jnp.dot(agg, wl_ref[...], preferred_element_type=jnp.float32)
         + bl_ref[...]
         + jnp.dot(h_ref[...], wr_ref[...], preferred_element_type=jnp.float32))
    if bn:
        mu = jnp.mean(z, axis=0, keepdims=True)
        var = jnp.mean((z - mu) ** 2, axis=0, keepdims=True)
        z = (z - mu) * lax.rsqrt(var + 1e-5) * g_ref[...] + b_ref[...]
        z = jnp.maximum(z, 0.0)
    out_ref[...] = z


def _tc_dense(bn, p, ccnt, h, wl, bl, wr, g, b):
    return pl.pallas_call(
        functools.partial(_tc_dense_body, bn),
        out_shape=jax.ShapeDtypeStruct((N, D), jnp.float32),
    )(p, ccnt, h, wl, bl.reshape(1, D), wr, g.reshape(1, D), b.reshape(1, D))


def kernel(x, edge_index, W_l0, b_l0, W_r0, W_l1, b_l1, W_r1, W_l2, b_l2,
           W_r2, bn_g0, bn_b0, bn_g1, bn_b1):
    src = edge_index[0].astype(jnp.int32)
    dst = edge_index[1].astype(jnp.int32)
    ar = jnp.arange(EPAD, dtype=jnp.int32)
    srcp = jnp.concatenate([src, ar % N])
    dstp = jnp.concatenate([dst, N + (ar % PADR)])
    srcA = srcp[:E0].reshape(NS, NG0, GRP, CH)
    dstA = dstp[:E0].reshape(NS, NG0, GRP, CH)
    if NG1 > 0:
        srcB = srcp[E0:].reshape(NS, NG1, GRP, CH)
        dstB = dstp[E0:].reshape(NS, NG1, GRP, CH)
    else:
        srcB = jnp.zeros((NS, 1, GRP, CH), jnp.int32)
        dstB = jnp.full((NS, 1, GRP, CH), N, jnp.int32)
    ones_nd = jnp.ones((N, D), jnp.float32)

    agg = _sc_agg()
    ccnt = agg(ones_nd, srcA, dstA, srcB, dstB)
    p0 = agg(x, srcA, dstA, srcB, dstB)
    h1 = _tc_dense(True, p0, ccnt, x, W_l0, b_l0, W_r0, bn_g0, bn_b0)
    p1 = agg(h1, srcA, dstA, srcB, dstB)
    h2 = _tc_dense(True, p1, ccnt, h1, W_l1, b_l1, W_r1, bn_g1, bn_b1)
    p2 = agg(h2, srcA, dstA, srcB, dstB)
    h3 = _tc_dense(False, p2, ccnt, h2, W_l2, b_l2, W_r2, bn_g0, bn_b0)
    return h3


# speedup vs baseline: 4.5541x; 1.1750x over previous
import functools

import jax
import jax.numpy as jnp
from jax import lax
from jax.experimental import pallas as pl
from jax.experimental.pallas import tpu as pltpu
from jax.experimental.pallas import tpu_sc as plsc

N = 10000
D = 128
E = 320000

NC = 2
NS = 16
NW = NC * NS
CH = 128
GRP = 8
NG0 = 10
NG1 = 10
E0 = NS * NG0 * GRP * CH
E1 = NS * NG1 * GRP * CH
EPAD = E0 + E1 - E
PADR = 512
ROWA = 624
ROWL = N - 15 * ROWA


@functools.lru_cache(maxsize=None)
def _mesh():
    return plsc.VectorSubcoreMesh(core_axis_name="c", subcore_axis_name="s",
                                  num_cores=NC, num_subcores=NS)


def _sc_agg_body(h_hbm, srcA_hbm, dstA_hbm, srcB_hbm, dstB_hbm, p_hbm,
                 src_g, dst_g, src_h, dst_h, rowsA, rowsB,
                 semA, semB, semIS0, semID0, semIS1, semID1, acc_h):
    c = lax.axis_index("c")
    s = lax.axis_index("s")

    zv = jnp.zeros((16,), jnp.float32)

    @pl.loop(0, CH)
    def _(i):
        for j in range(D // 16):
            rowsA[i, pl.ds(j * 16, 16)] = zv

    @pl.loop(0, 4)
    def _(k):
        pltpu.sync_copy(rowsA, acc_h.at[pl.ds(s * ROWA + k * CH, CH)])
    pltpu.sync_copy(rowsA.at[pl.ds(0, ROWA - 4 * CH)],
                    acc_h.at[pl.ds(s * ROWA + 4 * CH, ROWA - 4 * CH)])
    pltpu.sync_copy(rowsA.at[pl.ds(0, PADR // NS)],
                    acc_h.at[pl.ds(N + s * (PADR // NS), PADR // NS)])

    @pl.when(s == NS - 1)
    def _():
        pltpu.sync_copy(rowsA.at[pl.ds(0, N - 16 * ROWA)],
                        acc_h.at[pl.ds(16 * ROWA, N - 16 * ROWA)])

    plsc.subcore_barrier()

    def start(idx_row, buf, sem):
        pltpu.async_copy(h_hbm.at[idx_row], buf, sem)

    def wait(buf, sem):
        pltpu.make_async_copy(h_hbm.at[pl.ds(0, CH)], buf, sem).wait()

    bufs = (rowsA, rowsB)
    sems = (semA, semB)
    slots = ((src_g, dst_g, semIS0, semID0), (src_h, dst_h, semIS1, semID1))

    def run_edges(src_hbm, dst_hbm, ngroups):
        def idx_load(g, sync):
            sg, dg, sis, sid = slots[g % 2]
            if sync:
                pltpu.sync_copy(src_hbm.at[s, g], sg)
                pltpu.sync_copy(dst_hbm.at[s, g], dg)
            else:
                pltpu.async_copy(src_hbm.at[s, g], sg, sis)
                pltpu.async_copy(dst_hbm.at[s, g], dg, sid)

        def idx_wait(g):
            sg, dg, sis, sid = slots[g % 2]
            pltpu.make_async_copy(src_hbm.at[s, 0], sg, sis).wait()
            pltpu.make_async_copy(dst_hbm.at[s, 0], dg, sid).wait()

        idx_load(0, True)
        if ngroups > 1:
            idx_load(1, False)
        prev = None
        for j in range(ngroups * GRP):
            g, r = divmod(j, GRP)
            if r == 0 and g > 0:
                idx_wait(g)
            start(slots[g % 2][0].at[r], bufs[j % 2], sems[j % 2])
            if prev is not None:
                pg, pr, pp = prev
                wait(bufs[pp], sems[pp])
                pltpu.sync_copy(bufs[pp], acc_h.at[slots[pg % 2][1].at[pr]],
                                add=True)
                if pr == GRP - 1 and pg + 2 < ngroups:
                    idx_load(pg + 2, False)
            prev = (g, r, j % 2)
        pg, pr, pp = prev
        wait(bufs[pp], sems[pp])
        pltpu.sync_copy(bufs[pp], acc_h.at[slots[pg % 2][1].at[pr]], add=True)

    @pl.when(c == 0)
    def _():
        run_edges(srcA_hbm, dstA_hbm, NG0)

    @pl.when(c == 1)
    def _():
        run_edges(srcB_hbm, dstB_hbm, NG1)

    plsc.subcore_barrier()

    @pl.when(s < NS - 1)
    def _():
        pltpu.sync_copy(acc_h.at[pl.ds(s * ROWA, ROWA)],
                        p_hbm.at[c, pl.ds(s * ROWA, ROWA)])

    @pl.when(s == NS - 1)
    def _():
        pltpu.sync_copy(acc_h.at[pl.ds(15 * ROWA, ROWL)],
                        p_hbm.at[c, pl.ds(15 * ROWA, ROWL)])


@functools.lru_cache(maxsize=None)
def _sc_agg():
    return pl.kernel(
        _sc_agg_body,
        out_type=jax.ShapeDtypeStruct((NC, N, D), jnp.float32),
        mesh=_mesh(),
        scratch_types=(
            pltpu.VMEM((GRP, CH), jnp.int32),
            pltpu.VMEM((GRP, CH), jnp.int32),
            pltpu.VMEM((GRP, CH), jnp.int32),
            pltpu.VMEM((GRP, CH), jnp.int32),
            pltpu.VMEM((CH, D), jnp.float32),
            pltpu.VMEM((CH, D), jnp.float32),
            pltpu.SemaphoreType.DMA,
            pltpu.SemaphoreType.DMA,
            pltpu.SemaphoreType.DMA,
            pltpu.SemaphoreType.DMA,
            pltpu.SemaphoreType.DMA,
            pltpu.SemaphoreType.DMA,
            pltpu.VMEM_SHARED((N + PADR, D), jnp.float32),
        ),
    )


def _tc_dense_body(bn, p_ref, c_ref, h_ref, wl_ref, bl_ref, wr_ref,
                   g_ref, b_ref, out_ref):
    cnt = c_ref[0, :, 0:1] + c_ref[1, :, 0:1]
    inv = 1.0 / jnp.maximum(cnt, 1.0)
    agg = (p_ref[0] + p_ref[1]) * inv
    z = (jnp.dot(agg, wl_ref[...], preferred_element_type=jnp.float32)
         + bl_ref[...]
         + jnp.dot(h_ref[...], wr_ref[...], preferred_element_type=jnp.float32))
    if bn:
        mu = jnp.mean(z, axis=0, keepdims=True)
        var = jnp.mean((z - mu) ** 2, axis=0, keepdims=True)
        z = (z - mu) * lax.rsqrt(var + 1e-5) * g_ref[...] + b_ref[...]
        z = jnp.maximum(z, 0.0)
    out_ref[...] = z


def _tc_dense(bn, p, ccnt, h, wl, bl, wr, g, b):
    return pl.pallas_call(
        functools.partial(_tc_dense_body, bn),
        out_shape=jax.ShapeDtypeStruct((N, D), jnp.float32),
    )(p, ccnt, h, wl, bl.reshape(1, D), wr, g.reshape(1, D), b.reshape(1, D))


def kernel(x, edge_index, W_l0, b_l0, W_r0, W_l1, b_l1, W_r1, W_l2, b_l2,
           W_r2, bn_g0, bn_b0, bn_g1, bn_b1):
    src = edge_index[0].astype(jnp.int32)
    dst = edge_index[1].astype(jnp.int32)
    ar = jnp.arange(EPAD, dtype=jnp.int32)
    srcp = jnp.concatenate([src, ar % N])
    dstp = jnp.concatenate([dst, N + (ar % PADR)])
    srcA = srcp[:E0].reshape(NS, NG0, GRP, CH)
    dstA = dstp[:E0].reshape(NS, NG0, GRP, CH)
    if NG1 > 0:
        srcB = srcp[E0:].reshape(NS, NG1, GRP, CH)
        dstB = dstp[E0:].reshape(NS, NG1, GRP, CH)
    else:
        srcB = jnp.zeros((NS, 1, GRP, CH), jnp.int32)
        dstB = jnp.full((NS, 1, GRP, CH), N, jnp.int32)
    ones_nd = jnp.ones((N, D), jnp.float32)

    agg = _sc_agg()
    ccnt = agg(ones_nd, srcA, dstA, srcB, dstB)
    p0 = agg(x, srcA, dstA, srcB, dstB)
    h1 = _tc_dense(True, p0, ccnt, x, W_l0, b_l0, W_r0, bn_g0, bn_b0)
    p1 = agg(h1, srcA, dstA, srcB, dstB)
    h2 = _tc_dense(True, p1, ccnt, h1, W_l1, b_l1, W_r1, bn_g1, bn_b1)
    p2 = agg(h2, srcA, dstA, srcB, dstB)
    h3 = _tc_dense(False, p2, ccnt, h2, W_l2, b_l2, W_r2, bn_g0, bn_b0)
    return h3
